# Initial kernel scaffold; baseline (speedup 1.0000x reference)
import jax
import jax.numpy as jnp
from jax.experimental import pallas as pl


def _vq_kernel(x_ref, cb_ref, cw_ref, ce_ref, oh_ref):
    xb = x_ref[:, 0, :]
    cb = cb_ref[0]
    cross = jnp.dot(xb, cb.T, preferred_element_type=jnp.float32)
    x_sq = jnp.sum(xb * xb, axis=1, keepdims=True)
    c_sq = jnp.sum(cb * cb, axis=1)
    dist = x_sq - 2.0 * cross + c_sq[None, :]
    idx = jnp.argmin(dist, axis=1)
    k_iota = jax.lax.broadcasted_iota(jnp.int32, dist.shape, 1)
    onehot = (k_iota == idx[:, None]).astype(jnp.float32)
    oh_ref[:, 0, :] = onehot
    ce = jnp.dot(onehot, cb, preferred_element_type=jnp.float32)
    ce_ref[...] = ce
    cw_ref[...] = ce


def kernel(x, codebook):
    batch, embed = x.shape
    dim_codes, book_size, dim_embedding = codebook.shape
    xr = x.reshape(batch, dim_codes, dim_embedding)

    b_blk = 256
    grid = (batch // b_blk, dim_codes)

    cw, ce, oh = pl.pallas_call(
        _vq_kernel,
        grid=grid,
        in_specs=[
            pl.BlockSpec((b_blk, 1, dim_embedding), lambda b, c: (b, c, 0)),
            pl.BlockSpec((1, book_size, dim_embedding), lambda b, c: (c, 0, 0)),
        ],
        out_specs=[
            pl.BlockSpec((b_blk, dim_embedding), lambda b, c: (b, c)),
            pl.BlockSpec((b_blk, dim_embedding), lambda b, c: (b, c)),
            pl.BlockSpec((b_blk, 1, book_size), lambda b, c: (b, c, 0)),
        ],
        out_shape=[
            jax.ShapeDtypeStruct((batch, embed), jnp.float32),
            jax.ShapeDtypeStruct((batch, embed), jnp.float32),
            jax.ShapeDtypeStruct((batch, dim_codes, book_size), jnp.float32),
        ],
    )(xr, codebook)
    return (cw, ce, oh)


# speedup vs baseline: 1.9138x; 1.9138x over previous
import jax
import jax.numpy as jnp
from jax.experimental import pallas as pl

_B_BLK = 256
_C_BLK = 8


def _vq_kernel(x_ref, cbt_ref, cw_ref, ce_ref, oh_ref):
    d = 16
    book_size = cbt_ref.shape[1]
    for c in range(_C_BLK):
        xb = x_ref[:, c * d:(c + 1) * d]
        cbt = cbt_ref[c * d:(c + 1) * d, :]
        cross = jnp.dot(xb, cbt, preferred_element_type=jnp.float32)
        x_sq = jnp.sum(xb * xb, axis=1, keepdims=True)
        c_sq = jnp.sum(cbt * cbt, axis=0)
        dist = x_sq - 2.0 * cross + c_sq[None, :]
        idx = jnp.argmin(dist, axis=1)
        k_iota = jax.lax.broadcasted_iota(jnp.int32, dist.shape, 1)
        onehot = (k_iota == idx[:, None]).astype(jnp.float32)
        oh_ref[:, c * book_size:(c + 1) * book_size] = onehot
        ce = jax.lax.dot_general(onehot, cbt,
                                 dimension_numbers=(((1,), (1,)), ((), ())),
                                 preferred_element_type=jnp.float32)
        ce_ref[:, c * d:(c + 1) * d] = ce
        cw_ref[:, c * d:(c + 1) * d] = ce


def kernel(x, codebook):
    batch, embed = x.shape
    dim_codes, book_size, dim_embedding = codebook.shape
    cbt = codebook.transpose(0, 2, 1).reshape(dim_codes * dim_embedding, book_size)

    grid = (batch // _B_BLK, dim_codes // _C_BLK)
    cw, ce, oh = pl.pallas_call(
        _vq_kernel,
        grid=grid,
        in_specs=[
            pl.BlockSpec((_B_BLK, _C_BLK * dim_embedding), lambda b, c: (b, c)),
            pl.BlockSpec((_C_BLK * dim_embedding, book_size), lambda b, c: (c, 0)),
        ],
        out_specs=[
            pl.BlockSpec((_B_BLK, _C_BLK * dim_embedding), lambda b, c: (b, c)),
            pl.BlockSpec((_B_BLK, _C_BLK * dim_embedding), lambda b, c: (b, c)),
            pl.BlockSpec((_B_BLK, _C_BLK * book_size), lambda b, c: (b, c)),
        ],
        out_shape=[
            jax.ShapeDtypeStruct((batch, embed), jnp.float32),
            jax.ShapeDtypeStruct((batch, embed), jnp.float32),
            jax.ShapeDtypeStruct((batch, dim_codes * book_size), jnp.float32),
        ],
    )(x, cbt)
    return (cw, ce, oh.reshape(batch, dim_codes, book_size))
